# baseline (speedup 1.0000x reference)
import functools

import jax
import jax.numpy as jnp
from jax import lax
from jax.experimental import pallas as pl
from jax.experimental.pallas import tpu as pltpu
from jax.experimental.pallas import tpu_sc as plsc

NC = 2
NS = 16
NW = NC * NS
SEQ = 8192
TOK = SEQ // NW
CH = 128
NCH = TOK // CH
D = 50


def _sc_partials(x_hbm, emb_hbm, part_hbm, idx_v, rows_v, out_v, sem):
    wid = lax.axis_index("s") * NC + lax.axis_index("c")

    pltpu.sync_copy(x_hbm.at[pl.ds(wid * NCH, NCH)], idx_v)
    for j in range(NCH):
        pltpu.async_copy(emb_hbm.at[idx_v.at[j]], rows_v.at[j], sem)
    for j in range(NCH):
        pltpu.make_async_copy(emb_hbm.at[idx_v.at[j]], rows_v.at[j], sem).wait()

    lane = lax.iota(jnp.int32, 16)
    zero = jnp.zeros((16,), jnp.float32)
    one = jnp.ones((16,), jnp.float32)

    def body(i, carry):
        s0, s1, s2, s3, c0, c1, c2, c3 = carry
        j = i // CH
        r = i % CH
        r0 = rows_v[j, r, pl.ds(0, 16)]
        r1 = rows_v[j, r, pl.ds(16, 16)]
        r2 = rows_v[j, r, pl.ds(32, 16)]
        r3 = jnp.where(lane >= 14, rows_v[j, r, pl.ds(34, 16)], zero)
        s0 = s0 + r0
        s1 = s1 + r1
        s2 = s2 + r2
        s3 = s3 + r3
        c0 = c0 + jnp.where(r0 != 0.0, one, zero)
        c1 = c1 + jnp.where(r1 != 0.0, one, zero)
        c2 = c2 + jnp.where(r2 != 0.0, one, zero)
        c3 = c3 + jnp.where(r3 != 0.0, one, zero)
        return s0, s1, s2, s3, c0, c1, c2, c3

    acc = lax.fori_loop(0, TOK, body, (zero,) * 8)
    for p in range(8):
        out_v[pl.ds(p * 16, 16)] = acc[p]
    pltpu.sync_copy(out_v, part_hbm.at[wid])


@jax.jit
def _stage1(x2d, emb):
    mesh = plsc.VectorSubcoreMesh(core_axis_name="c", subcore_axis_name="s")
    f = pl.kernel(
        _sc_partials,
        out_type=jax.ShapeDtypeStruct((NW, 128), jnp.float32),
        mesh=mesh,
        scratch_types=[
            pltpu.VMEM((NCH, CH), jnp.int32),
            pltpu.VMEM((NCH, CH, D), jnp.float32),
            pltpu.VMEM((128,), jnp.float32),
            pltpu.SemaphoreType.DMA,
        ],
        compiler_params=pltpu.CompilerParams(use_tc_tiling_on_sc=False),
    )
    return f(x2d, emb)


def _tc_epilogue(part_ref, w64_ref, b8_ref, eye_ref, out_ref):
    total = jnp.sum(part_ref[...], axis=0, keepdims=True)
    sums = total[:, :64]
    cnts = total[:, 64:]
    sw = lax.dot_general(sums, w64_ref[...],
                         (((1,), (1,)), ((), ())),
                         preferred_element_type=jnp.float32)
    denom = cnts + jnp.where(cnts == 0.0, 1e-10, 0.0)
    recip = 1.0 / denom
    recip_col = lax.dot_general(eye_ref[...], recip,
                                (((1,), (1,)), ((), ())),
                                preferred_element_type=jnp.float32)
    out_ref[...] = recip_col * sw + b8_ref[...]


@jax.jit
def _stage2(part, w64, b8, eye):
    return pl.pallas_call(
        _tc_epilogue,
        out_shape=jax.ShapeDtypeStruct((64, 8), jnp.float32),
    )(part, w64, b8, eye)


def kernel(x, emb, W, b):
    x2d = x.reshape(NW * NCH, CH)
    part = _stage1(x2d, emb)

    w64 = jnp.zeros((8, 64), jnp.float32)
    w64 = w64.at[:3, :48].set(W[:, :48])
    w64 = w64.at[:3, 62].set(W[:, 48])
    w64 = w64.at[:3, 63].set(W[:, 49])
    b8 = jnp.zeros((1, 8), jnp.float32).at[0, :3].set(b)
    eye = jnp.eye(64, dtype=jnp.float32)

    y64 = _stage2(part, w64, b8, eye)
    y = jnp.concatenate([y64[:48, :3], y64[62:64, :3]], axis=0)
    return y[None]


# speedup vs baseline: 4.6332x; 4.6332x over previous
import functools

import jax
import jax.numpy as jnp
from jax import lax
from jax.experimental import pallas as pl
from jax.experimental.pallas import tpu as pltpu
from jax.experimental.pallas import tpu_sc as plsc

NC = 2
NS = 16
NW = NC * NS
SEQ = 8192
TOK = SEQ // NW
CH = 128
NCH = TOK // CH
D = 50


def _sc_partials(x_hbm, emb_hbm, part_hbm, idx_v, rows_v, out_v, sem):
    wid = lax.axis_index("s") * NC + lax.axis_index("c")

    pltpu.sync_copy(x_hbm.at[pl.ds(wid * TOK, TOK)], idx_v)

    def fire(c, _):
        vec = idx_v[pl.ds(c * 16, 16)]
        for k in range(16):
            pltpu.async_copy(
                emb_hbm.at[pl.ds(vec[k], 1)],
                rows_v.at[pl.ds(c * 16 + k, 1)],
                sem,
            )
        return 0

    lax.fori_loop(0, TOK // 16, fire, 0)
    pltpu.make_async_copy(emb_hbm.at[pl.ds(0, TOK)], rows_v, sem).wait()

    lane = lax.iota(jnp.int32, 16)
    zero = jnp.zeros((16,), jnp.float32)
    one = jnp.ones((16,), jnp.float32)

    def body(i, carry):
        s0, s1, s2, s3, c0, c1, c2, c3 = carry
        r0 = rows_v[i, pl.ds(0, 16)]
        r1 = rows_v[i, pl.ds(16, 16)]
        r2 = rows_v[i, pl.ds(32, 16)]
        r3 = jnp.where(lane >= 14, rows_v[i, pl.ds(34, 16)], zero)
        s0 = s0 + r0
        s1 = s1 + r1
        s2 = s2 + r2
        s3 = s3 + r3
        c0 = c0 + jnp.where(r0 != 0.0, one, zero)
        c1 = c1 + jnp.where(r1 != 0.0, one, zero)
        c2 = c2 + jnp.where(r2 != 0.0, one, zero)
        c3 = c3 + jnp.where(r3 != 0.0, one, zero)
        return s0, s1, s2, s3, c0, c1, c2, c3

    acc = lax.fori_loop(0, TOK, body, (zero,) * 8)
    for p in range(8):
        out_v[pl.ds(p * 16, 16)] = acc[p]
    pltpu.sync_copy(out_v, part_hbm.at[wid])


@jax.jit
def _stage1(x1d, emb):
    mesh = plsc.VectorSubcoreMesh(core_axis_name="c", subcore_axis_name="s")
    f = pl.kernel(
        _sc_partials,
        out_type=jax.ShapeDtypeStruct((NW, 128), jnp.float32),
        mesh=mesh,
        scratch_types=[
            pltpu.VMEM((TOK,), jnp.int32),
            pltpu.VMEM((TOK, D), jnp.float32),
            pltpu.VMEM((128,), jnp.float32),
            pltpu.SemaphoreType.DMA,
        ],
    )
    return f(x1d, emb)


def _tc_epilogue(part_ref, w64_ref, b8_ref, eye_ref, out_ref):
    total = jnp.sum(part_ref[...], axis=0, keepdims=True)
    sums = total[:, :64]
    cnts = total[:, 64:]
    sw = lax.dot_general(sums, w64_ref[...],
                         (((1,), (1,)), ((), ())),
                         preferred_element_type=jnp.float32)
    denom = cnts + jnp.where(cnts == 0.0, 1e-10, 0.0)
    recip = 1.0 / denom
    recip_col = lax.dot_general(eye_ref[...], recip,
                                (((1,), (1,)), ((), ())),
                                preferred_element_type=jnp.float32)
    out_ref[...] = recip_col * sw + b8_ref[...]


@jax.jit
def _stage2(part, w64, b8, eye):
    return pl.pallas_call(
        _tc_epilogue,
        out_shape=jax.ShapeDtypeStruct((64, 8), jnp.float32),
    )(part, w64, b8, eye)


def kernel(x, emb, W, b):
    part = _stage1(x.reshape(SEQ), emb)

    w64 = jnp.zeros((8, 64), jnp.float32)
    w64 = w64.at[:3, :48].set(W[:, :48])
    w64 = w64.at[:3, 62].set(W[:, 48])
    w64 = w64.at[:3, 63].set(W[:, 49])
    b8 = jnp.zeros((1, 8), jnp.float32).at[0, :3].set(b)
    eye = jnp.eye(64, dtype=jnp.float32)

    y64 = _stage2(part, w64, b8, eye)
    y = jnp.concatenate([y64[:48, :3], y64[62:64, :3]], axis=0)
    return y[None]
